# Initial kernel scaffold; baseline (speedup 1.0000x reference)
import jax
import jax.numpy as jnp
from jax.experimental import pallas as pl


def kernel(fov, batch_logit_prob, batch_top_k_prob, batch_action_idx, agent_current_pos, possible_actions, step):
    raise NotImplementedError("write your pallas kernel here")


# speedup vs baseline: 1.9378x; 1.9378x over previous
import functools

import jax
import jax.numpy as jnp
from jax import lax
from jax.experimental import pallas as pl
from jax.experimental.pallas import tpu as pltpu
from jax.experimental.pallas import tpu_sc as plsc

_B, _H, _W = 16384, 64, 64
_HW = _H * _W
_NC, _NS, _L = 2, 16, 16
_NW = _NC * _NS
_CPT = _B // _NW
_NCHUNK = _CPT // _L
_GCH = 128
_NG = _CPT // _GCH

_RW = 128
_RPB = _HW // _RW

_ENCODE_BLOCK = 1.0
_ENCODE_TARGET = 2.0
_ENCODE_START_STEP_IDX = 10.0

_TC_BS = 256


def _sc_body(fovrows, aidx, py, px, dy, dx,
             cy_o, cx_o, sidx_o, mask_o,
             aidx_v, py_v, px_v, dy_v, dx_v, ny_v, nx_v, ridx_v, rows_v,
             cy_v, cx_v, sidx_v, mask_v, sem):
    wid = lax.axis_index("s") * _NC + lax.axis_index("c")
    base = wid * _CPT
    pltpu.sync_copy(aidx.at[pl.ds(base, _CPT)], aidx_v)
    pltpu.sync_copy(py.at[pl.ds(base, _CPT)], py_v)
    pltpu.sync_copy(px.at[pl.ds(base, _CPT)], px_v)
    pltpu.sync_copy(dy, dy_v)
    pltpu.sync_copy(dx, dx_v)
    lane = lax.iota(jnp.int32, _L)
    for i in range(_NCHUNK):
        sl = pl.ds(i * _L, _L)
        a = aidx_v[sl]
        ay = plsc.load_gather(dy_v, [a])
        ax = plsc.load_gather(dx_v, [a])
        ny = jnp.clip(py_v[sl] + ay, 0, _H - 1)
        nx = jnp.clip(px_v[sl] + ax, 0, _W - 1)
        b = base + i * _L + lane
        ridx_v[sl] = b * _RPB + (ny >> 1)
        ny_v[sl] = ny
        nx_v[sl] = nx
    descs = []
    for j in range(_NG):
        descs.append(pltpu.async_copy(
            fovrows.at[ridx_v.at[pl.ds(j * _GCH, _GCH)]],
            rows_v.at[pl.ds(j * _GCH, _GCH)], sem))
    for d in descs:
        d.wait()
    for i in range(_NCHUNK):
        sl = pl.ds(i * _L, _L)
        r = i * _L + lane
        nx = nx_v[sl]
        off = ((ny_v[sl] & 1) << 6) + nx
        vals = plsc.load_gather(rows_v, [r, off])
        blocked = vals == jnp.float32(_ENCODE_BLOCK)
        target = vals == jnp.float32(_ENCODE_TARGET)
        cy = jnp.where(blocked, py_v[sl], ny_v[sl])
        cx = jnp.where(blocked, px_v[sl], nx)
        cy_v[sl] = cy
        cx_v[sl] = cx
        sidx_v[sl] = cy * _W + cx
        mask_v[sl] = jnp.where(target, jnp.int32(1), jnp.int32(0))
    pltpu.sync_copy(cy_v, cy_o.at[pl.ds(base, _CPT)])
    pltpu.sync_copy(cx_v, cx_o.at[pl.ds(base, _CPT)])
    pltpu.sync_copy(sidx_v, sidx_o.at[pl.ds(base, _CPT)])
    pltpu.sync_copy(mask_v, mask_o.at[pl.ds(base, _CPT)])


@functools.cache
def _make_sc_call():
  return functools.partial(
    pl.kernel,
    mesh=plsc.VectorSubcoreMesh(core_axis_name="c", subcore_axis_name="s"),
    compiler_params=pltpu.CompilerParams(needs_layout_passes=False),
    out_type=[jax.ShapeDtypeStruct((_B,), jnp.int32) for _ in range(4)],
    scratch_types=[
        pltpu.VMEM((_CPT,), jnp.int32),
        pltpu.VMEM((_CPT,), jnp.int32),
        pltpu.VMEM((_CPT,), jnp.int32),
        pltpu.VMEM((_L,), jnp.int32),
        pltpu.VMEM((_L,), jnp.int32),
        pltpu.VMEM((_CPT,), jnp.int32),
        pltpu.VMEM((_CPT,), jnp.int32),
        pltpu.VMEM((_CPT,), jnp.int32),
        pltpu.VMEM((_CPT, _RW), jnp.float32),
        pltpu.VMEM((_CPT,), jnp.int32),
        pltpu.VMEM((_CPT,), jnp.int32),
        pltpu.VMEM((_CPT,), jnp.int32),
        pltpu.VMEM((_CPT,), jnp.int32),
        pltpu.SemaphoreType.DMA,
    ],
  )(_sc_body)


def _tc_body(sidx_ref, marker_ref, fov_ref, out_ref):
    ids = sidx_ref[0]
    col = lax.broadcasted_iota(jnp.int32, (_TC_BS, _HW), 1)
    out_ref[...] = jnp.where(col == ids, marker_ref[0, 0], fov_ref[...])


def kernel(fov, batch_logit_prob, batch_top_k_prob, batch_action_idx,
           agent_current_pos, possible_actions, step):
    fovrows = fov.reshape(_B * _RPB, _RW)
    aidx = batch_action_idx.reshape(_B)
    py = agent_current_pos[:, 0]
    px = agent_current_pos[:, 1]
    dy16 = jnp.zeros((_L,), jnp.int32).at[:9].set(possible_actions[:, 0])
    dx16 = jnp.zeros((_L,), jnp.int32).at[:9].set(possible_actions[:, 1])

    cy, cx, sidx, mask_i32 = _make_sc_call()(fovrows, aidx, py, px, dy16, dx16)

    marker = (jnp.float32(_ENCODE_START_STEP_IDX)
              + jnp.asarray(step, jnp.float32)).reshape(1, 1)
    nb = _B // _TC_BS
    sidx3 = sidx.reshape(nb, _TC_BS, 1)
    new_fov = pl.pallas_call(
        _tc_body,
        grid=(nb,),
        in_specs=[
            pl.BlockSpec((1, _TC_BS, 1), lambda i: (i, 0, 0)),
            pl.BlockSpec(memory_space=pltpu.SMEM),
            pl.BlockSpec((_TC_BS, _HW), lambda i: (i, 0)),
        ],
        out_specs=pl.BlockSpec((_TC_BS, _HW), lambda i: (i, 0)),
        out_shape=jax.ShapeDtypeStruct((_B, _HW), jnp.float32),
    )(sidx3, marker, fov.reshape(_B, _HW)).reshape(_B, _H, _W)

    new_cur_pos = jnp.stack([cy, cx], axis=1)
    at_target = mask_i32.astype(jnp.bool_)
    return (new_fov, new_cur_pos, at_target,
            batch_action_idx, batch_logit_prob, batch_top_k_prob)


# speedup vs baseline: 11.7779x; 6.0780x over previous
import functools

import jax
import jax.numpy as jnp
from jax import lax
from jax.experimental import pallas as pl
from jax.experimental.pallas import tpu as pltpu
from jax.experimental.pallas import tpu_sc as plsc

_B, _H, _W = 16384, 64, 64
_HW = _H * _W
_NC, _NS, _L = 2, 16, 16
_NW = _NC * _NS
_CPT = _B // _NW
_NCHUNK = _CPT // _L
_GCH = 128
_NG = _CPT // _GCH

_ENCODE_BLOCK = 1.0
_ENCODE_TARGET = 2.0
_ENCODE_START_STEP_IDX = 10.0

_RBS = 64


def _sc_body(fovT, aidx, py, px, dy, dx,
             cy_o, cx_o, sidx_o, mask_o,
             aidx_v, py_v, px_v, dy_v, dx_v, ny_v, nx_v, cidx_v, rows_v,
             cy_v, cx_v, sidx_v, mask_v, sem):
    wid = lax.axis_index("s") * _NC + lax.axis_index("c")
    base = wid * _CPT
    pltpu.sync_copy(aidx.at[pl.ds(base, _CPT)], aidx_v)
    pltpu.sync_copy(py.at[pl.ds(base, _CPT)], py_v)
    pltpu.sync_copy(px.at[pl.ds(base, _CPT)], px_v)
    pltpu.sync_copy(dy, dy_v)
    pltpu.sync_copy(dx, dx_v)
    lane = lax.iota(jnp.int32, _L)
    for i in range(_NCHUNK):
        sl = pl.ds(i * _L, _L)
        a = aidx_v[sl]
        ay = plsc.load_gather(dy_v, [a])
        ax = plsc.load_gather(dx_v, [a])
        ny = jnp.clip(py_v[sl] + ay, 0, _H - 1)
        nx = jnp.clip(px_v[sl] + ax, 0, _W - 1)
        cidx_v[sl] = ny * _W + nx
        ny_v[sl] = ny
        nx_v[sl] = nx
    descs = []
    for g in range(_NG):
        b0 = pl.multiple_of(base + g * _GCH, _GCH)
        descs.append(pltpu.async_copy(
            fovT.at[cidx_v.at[pl.ds(g * _GCH, _GCH)], pl.ds(b0, _GCH)],
            rows_v.at[g], sem))
    for d in descs:
        d.wait()
    for i in range(_NCHUNK):
        sl = pl.ds(i * _L, _L)
        g16 = jnp.full((_L,), i // 8, jnp.int32)
        r = (i % 8) * _L + lane
        vals = plsc.load_gather(rows_v, [g16, r, r])
        blocked = vals == jnp.float32(_ENCODE_BLOCK)
        target = vals == jnp.float32(_ENCODE_TARGET)
        cy = jnp.where(blocked, py_v[sl], ny_v[sl])
        cx = jnp.where(blocked, px_v[sl], nx_v[sl])
        cy_v[sl] = cy
        cx_v[sl] = cx
        sidx_v[sl] = cy * _W + cx
        mask_v[sl] = jnp.where(target, jnp.int32(1), jnp.int32(0))
    pltpu.sync_copy(cy_v, cy_o.at[pl.ds(base, _CPT)])
    pltpu.sync_copy(cx_v, cx_o.at[pl.ds(base, _CPT)])
    pltpu.sync_copy(sidx_v, sidx_o.at[pl.ds(base, _CPT)])
    pltpu.sync_copy(mask_v, mask_o.at[pl.ds(base, _CPT)])


@functools.cache
def _make_sc_call():
  return functools.partial(
    pl.kernel,
    mesh=plsc.VectorSubcoreMesh(core_axis_name="c", subcore_axis_name="s"),
    compiler_params=pltpu.CompilerParams(needs_layout_passes=False),
    out_type=[jax.ShapeDtypeStruct((_B,), jnp.int32) for _ in range(4)],
    scratch_types=[
        pltpu.VMEM((_CPT,), jnp.int32),
        pltpu.VMEM((_CPT,), jnp.int32),
        pltpu.VMEM((_CPT,), jnp.int32),
        pltpu.VMEM((_L,), jnp.int32),
        pltpu.VMEM((_L,), jnp.int32),
        pltpu.VMEM((_CPT,), jnp.int32),
        pltpu.VMEM((_CPT,), jnp.int32),
        pltpu.VMEM((_CPT,), jnp.int32),
        pltpu.VMEM((_NG, _GCH, _GCH), jnp.float32),
        pltpu.VMEM((_CPT,), jnp.int32),
        pltpu.VMEM((_CPT,), jnp.int32),
        pltpu.VMEM((_CPT,), jnp.int32),
        pltpu.VMEM((_CPT,), jnp.int32),
        pltpu.SemaphoreType.DMA,
    ],
  )(_sc_body)


def _tc_body(sidx_ref, marker_ref, fov_ref, out_ref):
    ids = sidx_ref[...]
    rowid = (lax.broadcasted_iota(jnp.int32, (_RBS, _B), 0)
             + pl.program_id(0) * _RBS)
    out_ref[...] = jnp.where(rowid == ids, marker_ref[0, 0], fov_ref[...])


def kernel(fov, batch_logit_prob, batch_top_k_prob, batch_action_idx,
           agent_current_pos, possible_actions, step):
    fovT = fov.transpose(1, 2, 0).reshape(_HW, _B)
    aidx = batch_action_idx.reshape(_B)
    py = agent_current_pos[:, 0]
    px = agent_current_pos[:, 1]
    dy16 = jnp.zeros((_L,), jnp.int32).at[:9].set(possible_actions[:, 0])
    dx16 = jnp.zeros((_L,), jnp.int32).at[:9].set(possible_actions[:, 1])

    cy, cx, sidx, mask_i32 = _make_sc_call()(fovT, aidx, py, px, dy16, dx16)

    marker = (jnp.float32(_ENCODE_START_STEP_IDX)
              + jnp.asarray(step, jnp.float32)).reshape(1, 1)
    outT = pl.pallas_call(
        _tc_body,
        grid=(_HW // _RBS,),
        in_specs=[
            pl.BlockSpec((1, _B), lambda i: (0, 0)),
            pl.BlockSpec(memory_space=pltpu.SMEM),
            pl.BlockSpec((_RBS, _B), lambda i: (i, 0)),
        ],
        out_specs=pl.BlockSpec((_RBS, _B), lambda i: (i, 0)),
        out_shape=jax.ShapeDtypeStruct((_HW, _B), jnp.float32),
    )(sidx.reshape(1, _B), marker, fovT)
    new_fov = outT.reshape(_H, _W, _B).transpose(2, 0, 1)

    new_cur_pos = jnp.stack([cy, cx], axis=1)
    at_target = mask_i32.astype(jnp.bool_)
    return (new_fov, new_cur_pos, at_target,
            batch_action_idx, batch_logit_prob, batch_top_k_prob)


# speedup vs baseline: 11.8862x; 1.0092x over previous
import functools

import jax
import jax.numpy as jnp
from jax import lax
from jax.experimental import pallas as pl
from jax.experimental.pallas import tpu as pltpu
from jax.experimental.pallas import tpu_sc as plsc

_B, _H, _W = 16384, 64, 64
_HW = _H * _W
_NC, _NS, _L = 2, 16, 16
_NW = _NC * _NS
_CPT = _B // _NW
_NCHUNK = _CPT // _L
_GCH = 128
_NG = _CPT // _GCH

_ENCODE_BLOCK = 1.0
_ENCODE_TARGET = 2.0
_ENCODE_START_STEP_IDX = 10.0

_RBS = 128


def _sc_body(fovT, aidx, py, px, dy, dx,
             cy_o, cx_o, sidx_o, mask_o,
             aidx_v, py_v, px_v, dy_v, dx_v, ny_v, nx_v, cidx_v, rows_v,
             cy_v, cx_v, sidx_v, mask_v, sem):
    wid = lax.axis_index("s") * _NC + lax.axis_index("c")
    base = wid * _CPT
    pltpu.sync_copy(aidx.at[pl.ds(base, _CPT)], aidx_v)
    pltpu.sync_copy(py.at[pl.ds(base, _CPT)], py_v)
    pltpu.sync_copy(px.at[pl.ds(base, _CPT)], px_v)
    pltpu.sync_copy(dy, dy_v)
    pltpu.sync_copy(dx, dx_v)
    lane = lax.iota(jnp.int32, _L)
    for i in range(_NCHUNK):
        sl = pl.ds(i * _L, _L)
        a = aidx_v[sl]
        ay = plsc.load_gather(dy_v, [a])
        ax = plsc.load_gather(dx_v, [a])
        ny = jnp.clip(py_v[sl] + ay, 0, _H - 1)
        nx = jnp.clip(px_v[sl] + ax, 0, _W - 1)
        cidx_v[sl] = ny * _W + nx
        ny_v[sl] = ny
        nx_v[sl] = nx
    descs = []
    for g in range(_NG):
        b0 = pl.multiple_of(base + g * _GCH, _GCH)
        descs.append(pltpu.async_copy(
            fovT.at[cidx_v.at[pl.ds(g * _GCH, _GCH)], pl.ds(b0, _GCH)],
            rows_v.at[g], sem))
    for d in descs:
        d.wait()
    for i in range(_NCHUNK):
        sl = pl.ds(i * _L, _L)
        g16 = jnp.full((_L,), i // 8, jnp.int32)
        r = (i % 8) * _L + lane
        vals = plsc.load_gather(rows_v, [g16, r, r])
        blocked = vals == jnp.float32(_ENCODE_BLOCK)
        target = vals == jnp.float32(_ENCODE_TARGET)
        cy = jnp.where(blocked, py_v[sl], ny_v[sl])
        cx = jnp.where(blocked, px_v[sl], nx_v[sl])
        cy_v[sl] = cy
        cx_v[sl] = cx
        sidx_v[sl] = cy * _W + cx
        mask_v[sl] = jnp.where(target, jnp.int32(1), jnp.int32(0))
    pltpu.sync_copy(cy_v, cy_o.at[pl.ds(base, _CPT)])
    pltpu.sync_copy(cx_v, cx_o.at[pl.ds(base, _CPT)])
    pltpu.sync_copy(sidx_v, sidx_o.at[pl.ds(base, _CPT)])
    pltpu.sync_copy(mask_v, mask_o.at[pl.ds(base, _CPT)])


@functools.cache
def _make_sc_call():
  return functools.partial(
    pl.kernel,
    mesh=plsc.VectorSubcoreMesh(core_axis_name="c", subcore_axis_name="s"),
    compiler_params=pltpu.CompilerParams(needs_layout_passes=False),
    out_type=[jax.ShapeDtypeStruct((_B,), jnp.int32) for _ in range(4)],
    scratch_types=[
        pltpu.VMEM((_CPT,), jnp.int32),
        pltpu.VMEM((_CPT,), jnp.int32),
        pltpu.VMEM((_CPT,), jnp.int32),
        pltpu.VMEM((_L,), jnp.int32),
        pltpu.VMEM((_L,), jnp.int32),
        pltpu.VMEM((_CPT,), jnp.int32),
        pltpu.VMEM((_CPT,), jnp.int32),
        pltpu.VMEM((_CPT,), jnp.int32),
        pltpu.VMEM((_NG, _GCH, _GCH), jnp.float32),
        pltpu.VMEM((_CPT,), jnp.int32),
        pltpu.VMEM((_CPT,), jnp.int32),
        pltpu.VMEM((_CPT,), jnp.int32),
        pltpu.VMEM((_CPT,), jnp.int32),
        pltpu.SemaphoreType.DMA,
    ],
  )(_sc_body)


def _tc_body(sidx_ref, marker_ref, fov_ref, out_ref):
    ids = sidx_ref[...]
    rowid = (lax.broadcasted_iota(jnp.int32, (_RBS, _B), 0)
             + pl.program_id(0) * _RBS)
    out_ref[...] = jnp.where(rowid == ids, marker_ref[0, 0], fov_ref[...])


def kernel(fov, batch_logit_prob, batch_top_k_prob, batch_action_idx,
           agent_current_pos, possible_actions, step):
    fovT = fov.transpose(1, 2, 0).reshape(_HW, _B)
    aidx = batch_action_idx.reshape(_B)
    py = agent_current_pos[:, 0]
    px = agent_current_pos[:, 1]
    dy16 = jnp.zeros((_L,), jnp.int32).at[:9].set(possible_actions[:, 0])
    dx16 = jnp.zeros((_L,), jnp.int32).at[:9].set(possible_actions[:, 1])

    cy, cx, sidx, mask_i32 = _make_sc_call()(fovT, aidx, py, px, dy16, dx16)

    marker = (jnp.float32(_ENCODE_START_STEP_IDX)
              + jnp.asarray(step, jnp.float32)).reshape(1, 1)
    outT = pl.pallas_call(
        _tc_body,
        grid=(_HW // _RBS,),
        in_specs=[
            pl.BlockSpec((1, _B), lambda i: (0, 0)),
            pl.BlockSpec(memory_space=pltpu.SMEM),
            pl.BlockSpec((_RBS, _B), lambda i: (i, 0)),
        ],
        out_specs=pl.BlockSpec((_RBS, _B), lambda i: (i, 0)),
        out_shape=jax.ShapeDtypeStruct((_HW, _B), jnp.float32),
    )(sidx.reshape(1, _B), marker, fovT)
    new_fov = outT.reshape(_H, _W, _B).transpose(2, 0, 1)

    new_cur_pos = jnp.stack([cy, cx], axis=1)
    at_target = mask_i32.astype(jnp.bool_)
    return (new_fov, new_cur_pos, at_target,
            batch_action_idx, batch_logit_prob, batch_top_k_prob)
